# baseline (speedup 1.0000x reference)
import functools

import jax
import jax.numpy as jnp
from jax import lax
from jax.experimental import pallas as pl
from jax.experimental.pallas import tpu as pltpu
from jax.experimental.pallas import tpu_sc as plsc

VOCAB = 100000
EMB = 64
BATCH = 1024

_info = plsc.get_sparse_core_info()
_NC, _NS = _info.num_cores, _info.num_subcores
_NW = _NC * _NS
_B_PER_W = BATCH // _NW

_mesh = plsc.VectorSubcoreMesh(core_axis_name="c", subcore_axis_name="s")


@functools.partial(
    pl.kernel,
    mesh=_mesh,
    out_type=jax.ShapeDtypeStruct((BATCH, EMB), jnp.float32),
    scratch_types=[
        pltpu.VMEM((_B_PER_W,), jnp.int32),
        pltpu.VMEM((_B_PER_W, EMB), jnp.float32),
        pltpu.SemaphoreType.DMA,
    ],
    compiler_params=pltpu.CompilerParams(use_tc_tiling_on_sc=False),
)
def _sc_gather(table_hbm, idx_hbm, out_hbm, idx_v, rows_v, sem):
    wid = lax.axis_index("s") * _NC + lax.axis_index("c")
    base = wid * _B_PER_W
    pltpu.sync_copy(idx_hbm.at[pl.ds(base, _B_PER_W)], idx_v)
    pltpu.async_copy(table_hbm.at[idx_v], rows_v, sem).wait()
    pltpu.sync_copy(rows_v, out_hbm.at[pl.ds(base, _B_PER_W)])


_VT = 2048
_GRID = (VOCAB + _VT - 1) // _VT


def _linear_body(e_ref, w_ref, b_ref, out_ref):
    acc = lax.dot_general(
        e_ref[...], w_ref[...],
        (((1,), (1,)), ((), ())),
        preferred_element_type=jnp.float32,
    )
    out_ref[...] = acc + b_ref[...]


def _linear(e, W, b2d):
    return pl.pallas_call(
        _linear_body,
        grid=(_GRID,),
        in_specs=[
            pl.BlockSpec((BATCH, EMB), lambda i: (0, 0)),
            pl.BlockSpec((_VT, EMB), lambda i: (i, 0)),
            pl.BlockSpec((1, _VT), lambda i: (0, i)),
        ],
        out_specs=pl.BlockSpec((BATCH, _VT), lambda i: (0, i)),
        out_shape=jax.ShapeDtypeStruct((BATCH, VOCAB), jnp.float32),
        compiler_params=pltpu.CompilerParams(
            dimension_semantics=("arbitrary",),
        ),
    )(e, W, b2d)


def kernel(x, emb_table, W, b):
    x = x.astype(jnp.int32)
    e = _sc_gather(emb_table, x)
    logits = _linear(e, W, b.reshape(1, VOCAB))
    return (logits, e)


# speedup vs baseline: 3.6938x; 3.6938x over previous
import functools

import jax
import jax.numpy as jnp
from jax import lax
from jax.experimental import pallas as pl
from jax.experimental.pallas import tpu as pltpu
from jax.experimental.pallas import tpu_sc as plsc

VOCAB = 100000
EMB = 64
BATCH = 1024

_info = plsc.get_sparse_core_info()
_NC, _NS = _info.num_cores, _info.num_subcores
_NW = _NC * _NS
_D_PER_W = EMB // _NW

_mesh = plsc.VectorSubcoreMesh(core_axis_name="c", subcore_axis_name="s")


@functools.partial(
    pl.kernel,
    mesh=_mesh,
    out_type=jax.ShapeDtypeStruct((EMB, BATCH), jnp.float32),
    scratch_types=[
        pltpu.VMEM((VOCAB,), jnp.float32),
        pltpu.VMEM((BATCH,), jnp.int32),
        pltpu.VMEM((BATCH,), jnp.float32),
    ],
    compiler_params=pltpu.CompilerParams(needs_layout_passes=False),
)
def _sc_gather_t(tablet_hbm, idx_hbm, out_hbm, row_v, idx_v, et_v):
    wid = lax.axis_index("s") * _NC + lax.axis_index("c")
    pltpu.sync_copy(idx_hbm, idx_v)

    def _per_dim(k, _):
        d = wid * _D_PER_W + k
        pltpu.sync_copy(tablet_hbm.at[d], row_v)

        def _chunk(c, _):
            ivec = idx_v[pl.ds(c * 16, 16)]
            et_v[pl.ds(c * 16, 16)] = plsc.load_gather(row_v, [ivec])
            return _

        lax.fori_loop(0, BATCH // 16, _chunk, 0, unroll=8)
        pltpu.sync_copy(et_v, out_hbm.at[d])
        return _

    lax.fori_loop(0, _D_PER_W, _per_dim, 0)


_VT = 2048
_GRID = (VOCAB + _VT - 1) // _VT


def _linear_body(wt_ref, et_ref, b_ref, out_ref):
    acc = lax.dot_general(
        wt_ref[...], et_ref[...],
        (((0,), (0,)), ((), ())),
        preferred_element_type=jnp.float32,
    )
    bias = lax.broadcast_in_dim(b_ref[...], (_VT, BATCH), (0,))
    out_ref[...] = acc + bias


def _linear_t(wt, et, b):
    return pl.pallas_call(
        _linear_body,
        grid=(_GRID,),
        in_specs=[
            pl.BlockSpec((EMB, _VT), lambda i: (0, i)),
            pl.BlockSpec((EMB, BATCH), lambda i: (0, 0)),
            pl.BlockSpec((_VT,), lambda i: (i,)),
        ],
        out_specs=pl.BlockSpec((_VT, BATCH), lambda i: (i, 0)),
        out_shape=jax.ShapeDtypeStruct((VOCAB, BATCH), jnp.float32),
        compiler_params=pltpu.CompilerParams(
            dimension_semantics=("arbitrary",),
        ),
    )(wt, et, b)


def kernel(x, emb_table, W, b):
    x = x.astype(jnp.int32)
    tablet = jnp.swapaxes(emb_table, 0, 1)
    wt = jnp.swapaxes(W, 0, 1)
    et = _sc_gather_t(tablet, x)
    logits_t = _linear_t(wt, et, b)
    return (jnp.swapaxes(logits_t, 0, 1), jnp.swapaxes(et, 0, 1))


# speedup vs baseline: 3.7518x; 1.0157x over previous
import functools

import jax
import jax.numpy as jnp
from jax import lax
from jax.experimental import pallas as pl
from jax.experimental.pallas import tpu as pltpu
from jax.experimental.pallas import tpu_sc as plsc

VOCAB = 100000
EMB = 64
BATCH = 1024

_info = plsc.get_sparse_core_info()
_NC, _NS = _info.num_cores, _info.num_subcores
_NW = _NC * _NS
_D_PER_W = EMB // _NW

_mesh = plsc.VectorSubcoreMesh(core_axis_name="c", subcore_axis_name="s")


@functools.partial(
    pl.kernel,
    mesh=_mesh,
    out_type=jax.ShapeDtypeStruct((EMB, BATCH), jnp.float32),
    scratch_types=[
        pltpu.VMEM((VOCAB,), jnp.float32),
        pltpu.VMEM((BATCH,), jnp.int32),
        pltpu.VMEM((BATCH,), jnp.float32),
    ],
    compiler_params=pltpu.CompilerParams(needs_layout_passes=False),
)
def _sc_gather_t(tablet_hbm, idx_hbm, out_hbm, row_v, idx_v, et_v):
    wid = lax.axis_index("s") * _NC + lax.axis_index("c")
    pltpu.sync_copy(idx_hbm, idx_v)

    def _per_dim(k, _):
        d = wid * _D_PER_W + k
        pltpu.sync_copy(tablet_hbm.at[d], row_v)

        def _chunk(c, _):
            ivec = idx_v[pl.ds(c * 16, 16)]
            et_v[pl.ds(c * 16, 16)] = plsc.load_gather(row_v, [ivec])
            return _

        lax.fori_loop(0, BATCH // 16, _chunk, 0, unroll=8)
        pltpu.sync_copy(et_v, out_hbm.at[d])
        return _

    lax.fori_loop(0, _D_PER_W, _per_dim, 0)


_VT = 4096
_GRID = (VOCAB + _VT - 1) // _VT


def _linear_body(wt_ref, et_ref, b_ref, out_ref):
    acc = lax.dot_general(
        wt_ref[...], et_ref[...],
        (((0,), (0,)), ((), ())),
        preferred_element_type=jnp.float32,
    )
    bias = lax.broadcast_in_dim(b_ref[...], (_VT, BATCH), (0,))
    out_ref[...] = acc + bias


def _linear_t(wt, et, b):
    return pl.pallas_call(
        _linear_body,
        grid=(_GRID,),
        in_specs=[
            pl.BlockSpec((EMB, _VT), lambda i: (0, i)),
            pl.BlockSpec((EMB, BATCH), lambda i: (0, 0)),
            pl.BlockSpec((_VT,), lambda i: (i,)),
        ],
        out_specs=pl.BlockSpec((_VT, BATCH), lambda i: (i, 0)),
        out_shape=jax.ShapeDtypeStruct((VOCAB, BATCH), jnp.float32),
        compiler_params=pltpu.CompilerParams(
            dimension_semantics=("arbitrary",),
        ),
    )(wt, et, b)


def kernel(x, emb_table, W, b):
    x = x.astype(jnp.int32)
    tablet = jnp.swapaxes(emb_table, 0, 1)
    wt = jnp.swapaxes(W, 0, 1)
    et = _sc_gather_t(tablet, x)
    logits_t = _linear_t(wt, et, b)
    return (jnp.swapaxes(logits_t, 0, 1), jnp.swapaxes(et, 0, 1))
